# Initial kernel scaffold; baseline (speedup 1.0000x reference)
import jax
import jax.numpy as jnp
from jax.experimental import pallas as pl


def kernel(emotion_ids, table, W1, b1, W2, b2):
    raise NotImplementedError("write your pallas kernel here")


# speedup vs baseline: 3.6123x; 3.6123x over previous
import functools

import jax
import jax.numpy as jnp
from jax import lax
from jax.experimental import pallas as pl
from jax.experimental.pallas import tpu as pltpu
from jax.experimental.pallas import tpu_sc as plsc

_NC = 2
_NS = 16
_NW = _NC * _NS


def _mlp_body(tab_ref, w1_ref, b1_ref, w2_ref, b2_ref, out_ref):
    h = jnp.dot(tab_ref[...], w1_ref[...], preferred_element_type=jnp.float32)
    h = jnp.maximum(h + b1_ref[...], 0.0)
    o = jnp.dot(h, w2_ref[...], preferred_element_type=jnp.float32)
    out_ref[...] = o + b2_ref[...]


def _mlp_table(table, W1, b1, W2, b2):
    V, D = table.shape
    return pl.pallas_call(
        _mlp_body,
        out_shape=jax.ShapeDtypeStruct((V, D), jnp.float32),
    )(table, W1, b1.reshape(1, D), W2, b2.reshape(1, D))


@functools.lru_cache(maxsize=None)
def _make_gather(V, D, B, chunk):
    assert B % (_NW * chunk) == 0 and chunk % 8 == 0
    b_per_w = B // _NW
    n_chunks = b_per_w // chunk
    mesh = plsc.VectorSubcoreMesh(
        core_axis_name="c", subcore_axis_name="s",
        num_cores=_NC, num_subcores=_NS,
    )

    @functools.partial(
        pl.kernel,
        mesh=mesh,
        out_type=jax.ShapeDtypeStruct((B, D), jnp.float32),
        compiler_params=pltpu.CompilerParams(use_tc_tiling_on_sc=False),
        scratch_types=[
            pltpu.VMEM((chunk,), jnp.int32),
            pltpu.VMEM((chunk, D), jnp.float32),
            pltpu.SemaphoreType.DMA,
        ],
    )
    def gather(tab_hbm, idx_hbm, out_hbm, idx_v, rows_v, sem):
        wid = lax.axis_index("s") * _NC + lax.axis_index("c")
        base = wid * b_per_w

        def step(i, carry):
            off = base + i * chunk
            pltpu.sync_copy(idx_hbm.at[pl.ds(off, chunk)], idx_v)
            pltpu.async_copy(tab_hbm.at[idx_v], rows_v, sem).wait()
            pltpu.sync_copy(rows_v, out_hbm.at[pl.ds(off, chunk)])
            return carry

        lax.fori_loop(0, n_chunks, step, 0)

    return gather


def kernel(emotion_ids, table, W1, b1, W2, b2):
    Bb, H = emotion_ids.shape
    V, D = table.shape
    mlp_tab = _mlp_table(table, W1, b1, W2, b2)
    flat_idx = emotion_ids.reshape(-1).astype(jnp.int32)
    out_flat = _make_gather(V, D, Bb * H, 1024)(mlp_tab, flat_idx)
    return out_flat.reshape(Bb, H, D)
